# baseline (speedup 1.0000x reference)
import functools

import jax
import jax.numpy as jnp
from jax import lax
from jax.experimental import pallas as pl
from jax.experimental.pallas import tpu as pltpu
from jax.experimental.pallas import tpu_sc as plsc

B = 16384
D = 64
V = 1000000
NC = 2
NS = 16
NW = NC * NS
BPW = B // NW
L = 16
NGRP = BPW // L

_mesh = plsc.VectorSubcoreMesh(core_axis_name="c", subcore_axis_name="s")


@functools.partial(
    pl.kernel,
    mesh=_mesh,
    compiler_params=pltpu.CompilerParams(
        needs_layout_passes=False, use_tc_tiling_on_sc=False),
    out_type=jax.ShapeDtypeStruct((B,), jnp.float32),
    scratch_types=[
        pltpu.VMEM((BPW,), jnp.int32),
        pltpu.VMEM((BPW,), jnp.int32),
        pltpu.VMEM((2, L, D), jnp.float32),
        pltpu.VMEM((2, L, D), jnp.float32),
        pltpu.VMEM((BPW,), jnp.float32),
        pltpu.SemaphoreType.DMA,
        pltpu.SemaphoreType.DMA,
        pltpu.SemaphoreType.DMA,
        pltpu.SemaphoreType.DMA,
    ],
)
def _pointmf_sc(user_hbm, item_hbm, eu_hbm, ei_hbm, out_hbm,
                uidx, iidx, ubuf, ibuf, outv, su0, su1, si0, si1):
    wid = lax.axis_index("s") * NC + lax.axis_index("c")
    base = wid * BPW

    pltpu.sync_copy(user_hbm.at[pl.ds(base, BPW)], uidx)
    pltpu.sync_copy(item_hbm.at[pl.ds(base, BPW)], iidx)

    lanes = lax.iota(jnp.int32, L)
    sems = (su0, su1, si0, si1)

    def fire(g, slot):
        r0 = g * L
        ou = uidx[pl.ds(r0, L)] << 6
        oi = iidx[pl.ds(r0, L)] << 6
        for c in range(L):
            pltpu.async_copy(
                eu_hbm.at[pl.ds(pl.multiple_of(ou[c], 64), D)],
                ubuf.at[slot, c], sems[slot])
            pltpu.async_copy(
                ei_hbm.at[pl.ds(pl.multiple_of(oi[c], 64), D)],
                ibuf.at[slot, c], sems[2 + slot])

    def wait(g, slot):
        for c in range(L):
            pltpu.make_async_copy(
                eu_hbm.at[pl.ds(0, D)], ubuf.at[slot, 0], sems[slot]).wait()
            pltpu.make_async_copy(
                ei_hbm.at[pl.ds(0, D)], ibuf.at[slot, 0], sems[2 + slot]).wait()

    def compute(g, slot):
        r0 = g * L
        out_vec = jnp.zeros((L,), jnp.float32)
        for c in range(L):
            acc = None
            for k in range(D // L):
                u = ubuf[slot, c, pl.ds(k * L, L)]
                v = ibuf[slot, c, pl.ds(k * L, L)]
                p = u * v
                acc = p if acc is None else acc + p
            csum = plsc.cumsum(acc)
            bs = lax.broadcast(csum[L - 1], (L,))
            out_vec = jnp.where(lanes == c, bs, out_vec)
        outv[pl.ds(r0, L)] = out_vec

    fire(0, 0)

    def pair_body(p, carry):
        g0 = p * 2
        fire(g0 + 1, 1)
        wait(g0, 0)
        compute(g0, 0)

        @pl.when(g0 + 2 < NGRP)
        def _():
            fire(g0 + 2, 0)

        wait(g0 + 1, 1)
        compute(g0 + 1, 1)
        return carry

    lax.fori_loop(0, NGRP // 2, pair_body, 0)
    pltpu.sync_copy(outv, out_hbm.at[pl.ds(base, BPW)])


def kernel(user, item, embed_user, embed_item):
    eu1 = embed_user.reshape(V * D)
    ei1 = embed_item.reshape(V * D)
    return _pointmf_sc(user, item, eu1, ei1)


# speedup vs baseline: 1.4839x; 1.4839x over previous
import functools

import jax
import jax.numpy as jnp
from jax import lax
from jax.experimental import pallas as pl
from jax.experimental.pallas import tpu as pltpu
from jax.experimental.pallas import tpu_sc as plsc

B = 16384
D = 64
V = 1000000
SUB = 8
NC = 2
NS = 16
NW = NC * NS
BPW = B // NW
L = 16
NGRP = BPW // L

_mesh = plsc.VectorSubcoreMesh(core_axis_name="c", subcore_axis_name="s")


@functools.partial(
    pl.kernel,
    mesh=_mesh,
    compiler_params=pltpu.CompilerParams(needs_layout_passes=False),
    out_type=jax.ShapeDtypeStruct((B,), jnp.float32),
    scratch_types=[
        pltpu.VMEM((BPW,), jnp.int32),
        pltpu.VMEM((BPW,), jnp.int32),
        pltpu.VMEM((2, L, SUB, D), jnp.float32),
        pltpu.VMEM((2, L, SUB, D), jnp.float32),
        pltpu.VMEM((BPW,), jnp.float32),
        pltpu.SemaphoreType.DMA,
        pltpu.SemaphoreType.DMA,
        pltpu.SemaphoreType.DMA,
        pltpu.SemaphoreType.DMA,
    ],
)
def _pointmf_sc(user_hbm, item_hbm, eu_hbm, ei_hbm, out_hbm,
                uidx, iidx, ubuf, ibuf, outv, su0, su1, si0, si1):
    wid = lax.axis_index("s") * NC + lax.axis_index("c")
    base = wid * BPW

    pltpu.sync_copy(user_hbm.at[pl.ds(base, BPW)], uidx)
    pltpu.sync_copy(item_hbm.at[pl.ds(base, BPW)], iidx)

    lanes = lax.iota(jnp.int32, L)
    sems = (su0, su1, si0, si1)

    def fire(g, slot):
        r0 = g * L
        ou = (uidx[pl.ds(r0, L)] >> 3) << 3
        oi = (iidx[pl.ds(r0, L)] >> 3) << 3
        for c in range(L):
            pltpu.async_copy(
                eu_hbm.at[pl.ds(pl.multiple_of(ou[c], SUB), SUB)],
                ubuf.at[slot, c], sems[slot])
            pltpu.async_copy(
                ei_hbm.at[pl.ds(pl.multiple_of(oi[c], SUB), SUB)],
                ibuf.at[slot, c], sems[2 + slot])

    def wait(g, slot):
        for c in range(L):
            pltpu.make_async_copy(
                eu_hbm.at[pl.ds(0, SUB)], ubuf.at[slot, 0], sems[slot]).wait()
            pltpu.make_async_copy(
                ei_hbm.at[pl.ds(0, SUB)], ibuf.at[slot, 0],
                sems[2 + slot]).wait()

    def compute(g, slot):
        r0 = g * L
        su = uidx[pl.ds(r0, L)] & 7
        si = iidx[pl.ds(r0, L)] & 7
        out_vec = jnp.zeros((L,), jnp.float32)
        for c in range(L):
            ju = su[c]
            ji = si[c]
            acc = None
            for k in range(D // L):
                u = ubuf[slot, c, ju, pl.ds(k * L, L)]
                v = ibuf[slot, c, ji, pl.ds(k * L, L)]
                p = u * v
                acc = p if acc is None else acc + p
            csum = plsc.cumsum(acc)
            bs = lax.broadcast(csum[L - 1], (L,))
            out_vec = jnp.where(lanes == c, bs, out_vec)
        outv[pl.ds(r0, L)] = out_vec

    fire(0, 0)

    def pair_body(p, carry):
        g0 = p * 2
        fire(g0 + 1, 1)
        wait(g0, 0)
        compute(g0, 0)

        @pl.when(g0 + 2 < NGRP)
        def _():
            fire(g0 + 2, 0)

        wait(g0 + 1, 1)
        compute(g0 + 1, 1)
        return carry

    lax.fori_loop(0, NGRP // 2, pair_body, 0)
    pltpu.sync_copy(outv, out_hbm.at[pl.ds(base, BPW)])


def kernel(user, item, embed_user, embed_item):
    return _pointmf_sc(user, item, embed_user, embed_item)
